# Initial kernel scaffold; baseline (speedup 1.0000x reference)
import jax
import jax.numpy as jnp
from jax.experimental import pallas as pl


def kernel(feat, edge_index, etypes, W_lin, b_lin, W_ih, W_hh, b_ih, b_hh):
    raise NotImplementedError("write your pallas kernel here")


# speedup vs baseline: 14.6988x; 14.6988x over previous
import functools

import jax
import jax.numpy as jnp
from jax import lax
from jax.experimental import pallas as pl
from jax.experimental.pallas import tpu as pltpu
from jax.experimental.pallas import tpu_sc as plsc

N_NODES = 10000
N_EDGES = 320000
FEATS = 128
HF = FEATS // 2
N_STEPS = 3
N_ETYPES = 4

NC = 2
NS = 16
CH = 128
NCH = 160
E_PAD = NS * NCH * CH
RPT = 640
A_ROWS = NS * RPT

BN = 2000


def _trans_body(h_ref, wt_ref, b_ref, out_ref):
    res = (
        jnp.dot(h_ref[...], wt_ref[0], preferred_element_type=jnp.float32)
        + b_ref[0]
    )
    out_ref[0, 0] = res[:, :HF]
    out_ref[1, 0] = res[:, HF:]


def _trans_call(h, wt, b3):
    return pl.pallas_call(
        _trans_body,
        grid=(N_ETYPES, N_NODES // BN),
        in_specs=[
            pl.BlockSpec((BN, FEATS), lambda k, i: (i, 0)),
            pl.BlockSpec((1, FEATS, FEATS), lambda k, i: (k, 0, 0)),
            pl.BlockSpec((1, 1, FEATS), lambda k, i: (k, 0, 0)),
        ],
        out_specs=pl.BlockSpec((NC, 1, BN, HF), lambda k, i: (0, k, i, 0)),
        out_shape=jax.ShapeDtypeStruct(
            (NC, N_ETYPES, N_NODES, HF), jnp.float32),
    )(h, wt, b3)


def _sc_body(table_hbm, gidx_hbm, dst_hbm, zeros_hbm, out_hbm,
             gidx_v, dst_v, buf_a, buf_b, acc_sh, sem_a, sem_b):
    c = lax.axis_index("c")
    s = lax.axis_index("s")
    my_table = table_hbm.at[c]

    pltpu.sync_copy(gidx_hbm.at[s], gidx_v)
    pltpu.sync_copy(dst_hbm.at[s], dst_v)
    pltpu.sync_copy(zeros_hbm.at[s], acc_sh.at[pl.ds(s * RPT, RPT)])
    plsc.subcore_barrier()

    def body(jj, carry):
        j = jj * 2
        cp_a = pltpu.async_copy(my_table.at[gidx_v.at[j]], buf_a, sem_a)
        cp_b = pltpu.async_copy(my_table.at[gidx_v.at[j + 1]], buf_b, sem_b)
        cp_a.wait()
        pltpu.sync_copy(buf_a, acc_sh.at[dst_v.at[j]], add=True)
        cp_b.wait()
        pltpu.sync_copy(buf_b, acc_sh.at[dst_v.at[j + 1]], add=True)
        return carry

    lax.fori_loop(0, NCH // 2, body, 0)
    plsc.subcore_barrier()

    pltpu.sync_copy(acc_sh.at[pl.ds(s * RPT, RPT)], out_hbm.at[c, s])


@functools.cache
def _sc_aggregate_fn():
    mesh = plsc.VectorSubcoreMesh(core_axis_name="c", subcore_axis_name="s")
    return pl.kernel(
        _sc_body,
        mesh=mesh,
        compiler_params=pltpu.CompilerParams(use_tc_tiling_on_sc=False),
        out_type=jax.ShapeDtypeStruct((NC, NS, RPT, HF), jnp.float32),
        scratch_types=[
            pltpu.VMEM((NCH, CH), jnp.int32),
            pltpu.VMEM((NCH, CH), jnp.int32),
            pltpu.VMEM((CH, HF), jnp.float32),
            pltpu.VMEM((CH, HF), jnp.float32),
            pltpu.VMEM_SHARED((A_ROWS, HF), jnp.float32),
            pltpu.SemaphoreType.DMA,
            pltpu.SemaphoreType.DMA,
        ],
    )


def _gru_body(ap_ref, h_ref, wih_ref, whh_ref, bih_ref, bhh_ref, out_ref):
    a = jnp.concatenate([ap_ref[0], ap_ref[1]], axis=-1)
    h = h_ref[...]
    gi = jnp.dot(a, wih_ref[...], preferred_element_type=jnp.float32) + bih_ref[...]
    gh = jnp.dot(h, whh_ref[...], preferred_element_type=jnp.float32) + bhh_ref[...]
    r = jax.nn.sigmoid(gi[:, 0:FEATS] + gh[:, 0:FEATS])
    z = jax.nn.sigmoid(gi[:, FEATS:2 * FEATS] + gh[:, FEATS:2 * FEATS])
    n = jnp.tanh(gi[:, 2 * FEATS:] + r * gh[:, 2 * FEATS:])
    out_ref[...] = (1.0 - z) * n + z * h


def _gru_call(parts, h, wih_t, whh_t, bih2, bhh2):
    return pl.pallas_call(
        _gru_body,
        grid=(N_NODES // BN,),
        in_specs=[
            pl.BlockSpec((NC, BN, HF), lambda i: (0, i, 0)),
            pl.BlockSpec((BN, FEATS), lambda i: (i, 0)),
            pl.BlockSpec((FEATS, 3 * FEATS), lambda i: (0, 0)),
            pl.BlockSpec((FEATS, 3 * FEATS), lambda i: (0, 0)),
            pl.BlockSpec((1, 3 * FEATS), lambda i: (0, 0)),
            pl.BlockSpec((1, 3 * FEATS), lambda i: (0, 0)),
        ],
        out_specs=pl.BlockSpec((BN, FEATS), lambda i: (i, 0)),
        out_shape=jax.ShapeDtypeStruct((N_NODES, FEATS), jnp.float32),
    )(parts, h, wih_t, whh_t, bih2, bhh2)


def kernel(feat, edge_index, etypes, W_lin, b_lin, W_ih, W_hh, b_ih, b_hh):
    src = edge_index[0]
    dst = edge_index[1]

    pad = E_PAD - N_EDGES
    gidx = etypes * N_NODES + src
    gidx = jnp.concatenate([gidx, jnp.zeros((pad,), jnp.int32)])
    dstp = jnp.concatenate([dst, jnp.full((pad,), N_NODES, jnp.int32)])
    gidx3 = gidx.reshape(NS, NCH, CH)
    dst3 = dstp.reshape(NS, NCH, CH)
    zeros = jnp.zeros((NS, RPT, HF), jnp.float32)

    wt = W_lin.transpose(0, 2, 1)
    b3 = b_lin.reshape(N_ETYPES, 1, FEATS)
    wih_t = W_ih.T
    whh_t = W_hh.T
    bih2 = b_ih.reshape(1, 3 * FEATS)
    bhh2 = b_hh.reshape(1, 3 * FEATS)

    h = feat
    for _ in range(N_STEPS):
        trans = _trans_call(h, wt, b3)
        parts = _sc_aggregate_fn()(
            trans.reshape(NC, N_ETYPES * N_NODES, HF), gidx3, dst3, zeros)
        parts = parts.reshape(NC, A_ROWS, HF)
        h = _gru_call(parts, h, wih_t, whh_t, bih2, bhh2)
    return h


# speedup vs baseline: 16.6035x; 1.1296x over previous
import functools

import jax
import jax.numpy as jnp
from jax import lax
from jax.experimental import pallas as pl
from jax.experimental.pallas import tpu as pltpu
from jax.experimental.pallas import tpu_sc as plsc

N_NODES = 10000
N_EDGES = 320000
FEATS = 128
HF = FEATS // 2
N_STEPS = 3
N_ETYPES = 4

NC = 2
NS = 16
CH = 128
NCH = 160
E_PAD = NS * NCH * CH
RPT = 640
A_ROWS = NS * RPT

BN = 2000


def _trans_body(h_ref, wt_ref, b_ref, out_ref):
    res = (
        jnp.dot(h_ref[...], wt_ref[0], preferred_element_type=jnp.float32)
        + b_ref[0]
    )
    out_ref[0, 0] = res[:, :HF]
    out_ref[1, 0] = res[:, HF:]


def _trans_call(h, wt, b3):
    return pl.pallas_call(
        _trans_body,
        grid=(N_ETYPES, N_NODES // BN),
        in_specs=[
            pl.BlockSpec((BN, FEATS), lambda k, i: (i, 0)),
            pl.BlockSpec((1, FEATS, FEATS), lambda k, i: (k, 0, 0)),
            pl.BlockSpec((1, 1, FEATS), lambda k, i: (k, 0, 0)),
        ],
        out_specs=pl.BlockSpec((NC, 1, BN, HF), lambda k, i: (0, k, i, 0)),
        out_shape=jax.ShapeDtypeStruct(
            (NC, N_ETYPES, N_NODES, HF), jnp.float32),
    )(h, wt, b3)


NBUF = 4


def _sc_body(table_hbm, gidx_hbm, dst_hbm, zeros_hbm, out_hbm,
             gidx_v, dst_v, buf, acc_sh, gsem, asem):
    c = lax.axis_index("c")
    s = lax.axis_index("s")
    my_table = table_hbm.at[c]

    pltpu.sync_copy(gidx_hbm.at[s], gidx_v)
    pltpu.sync_copy(dst_hbm.at[s], dst_v)
    pltpu.sync_copy(zeros_hbm.at[s], acc_sh.at[pl.ds(s * RPT, RPT)])
    plsc.subcore_barrier()

    def gather_start(j, b):
        pltpu.async_copy(my_table.at[gidx_v.at[j]], buf.at[b], gsem.at[b])

    def gather_wait(j, b):
        pltpu.make_async_copy(
            my_table.at[gidx_v.at[j]], buf.at[b], gsem.at[b]).wait()

    def add_start(j, b):
        pltpu.async_copy(buf.at[b], acc_sh.at[dst_v.at[j]], asem.at[b],
                         add=True)

    def add_wait(j, b):
        pltpu.make_async_copy(
            buf.at[b], acc_sh.at[dst_v.at[j]], asem.at[b]).wait()

    for b in range(NBUF):
        gather_start(b, b)

    def round_body(jj, carry):
        base = jj * NBUF
        for b in range(NBUF):
            gather_wait(base + b, b)
            add_start(base + b, b)
        for b in range(NBUF):
            add_wait(base + b, b)
            gather_start(base + NBUF + b, b)
        return carry

    lax.fori_loop(0, NCH // NBUF - 1, round_body, 0)
    base = NCH - NBUF
    for b in range(NBUF):
        gather_wait(base + b, b)
        add_start(base + b, b)
    for b in range(NBUF):
        add_wait(base + b, b)
    plsc.subcore_barrier()

    pltpu.sync_copy(acc_sh.at[pl.ds(s * RPT, RPT)], out_hbm.at[c, s])


@functools.cache
def _sc_aggregate_fn():
    mesh = plsc.VectorSubcoreMesh(core_axis_name="c", subcore_axis_name="s")
    return pl.kernel(
        _sc_body,
        mesh=mesh,
        compiler_params=pltpu.CompilerParams(use_tc_tiling_on_sc=False),
        out_type=jax.ShapeDtypeStruct((NC, NS, RPT, HF), jnp.float32),
        scratch_types=[
            pltpu.VMEM((NCH, CH), jnp.int32),
            pltpu.VMEM((NCH, CH), jnp.int32),
            pltpu.VMEM((NBUF, CH, HF), jnp.float32),
            pltpu.VMEM_SHARED((A_ROWS, HF), jnp.float32),
            pltpu.SemaphoreType.DMA((NBUF,)),
            pltpu.SemaphoreType.DMA((NBUF,)),
        ],
    )


def _gru_body(ap_ref, h_ref, wih_ref, whh_ref, bih_ref, bhh_ref, out_ref):
    a = jnp.concatenate([ap_ref[0], ap_ref[1]], axis=-1)
    h = h_ref[...]
    gi = jnp.dot(a, wih_ref[...], preferred_element_type=jnp.float32) + bih_ref[...]
    gh = jnp.dot(h, whh_ref[...], preferred_element_type=jnp.float32) + bhh_ref[...]
    r = jax.nn.sigmoid(gi[:, 0:FEATS] + gh[:, 0:FEATS])
    z = jax.nn.sigmoid(gi[:, FEATS:2 * FEATS] + gh[:, FEATS:2 * FEATS])
    n = jnp.tanh(gi[:, 2 * FEATS:] + r * gh[:, 2 * FEATS:])
    out_ref[...] = (1.0 - z) * n + z * h


def _gru_call(parts, h, wih_t, whh_t, bih2, bhh2):
    return pl.pallas_call(
        _gru_body,
        grid=(N_NODES // BN,),
        in_specs=[
            pl.BlockSpec((NC, BN, HF), lambda i: (0, i, 0)),
            pl.BlockSpec((BN, FEATS), lambda i: (i, 0)),
            pl.BlockSpec((FEATS, 3 * FEATS), lambda i: (0, 0)),
            pl.BlockSpec((FEATS, 3 * FEATS), lambda i: (0, 0)),
            pl.BlockSpec((1, 3 * FEATS), lambda i: (0, 0)),
            pl.BlockSpec((1, 3 * FEATS), lambda i: (0, 0)),
        ],
        out_specs=pl.BlockSpec((BN, FEATS), lambda i: (i, 0)),
        out_shape=jax.ShapeDtypeStruct((N_NODES, FEATS), jnp.float32),
    )(parts, h, wih_t, whh_t, bih2, bhh2)


def kernel(feat, edge_index, etypes, W_lin, b_lin, W_ih, W_hh, b_ih, b_hh):
    src = edge_index[0]
    dst = edge_index[1]

    pad = E_PAD - N_EDGES
    gidx = etypes * N_NODES + src
    gidx = jnp.concatenate([gidx, jnp.zeros((pad,), jnp.int32)])
    dstp = jnp.concatenate([dst, jnp.full((pad,), N_NODES, jnp.int32)])
    gidx3 = gidx.reshape(NS, NCH, CH)
    dst3 = dstp.reshape(NS, NCH, CH)
    zeros = jnp.zeros((NS, RPT, HF), jnp.float32)

    wt = W_lin.transpose(0, 2, 1)
    b3 = b_lin.reshape(N_ETYPES, 1, FEATS)
    wih_t = W_ih.T
    whh_t = W_hh.T
    bih2 = b_ih.reshape(1, 3 * FEATS)
    bhh2 = b_hh.reshape(1, 3 * FEATS)

    def step(h, _):
        trans = _trans_call(h, wt, b3)
        parts = _sc_aggregate_fn()(
            trans.reshape(NC, N_ETYPES * N_NODES, HF), gidx3, dst3, zeros)
        parts = parts.reshape(NC, A_ROWS, HF)
        h = _gru_call(parts, h, wih_t, whh_t, bih2, bhh2)
        return h, None

    h, _ = lax.scan(step, feat, None, length=N_STEPS)
    return h


# speedup vs baseline: 19.2896x; 1.1618x over previous
import functools

import jax
import jax.numpy as jnp
from jax import lax
from jax.experimental import pallas as pl
from jax.experimental.pallas import tpu as pltpu
from jax.experimental.pallas import tpu_sc as plsc

N_NODES = 10000
N_EDGES = 320000
FEATS = 128
HF = FEATS // 2
PW = HF // 2
N_STEPS = 3
N_ETYPES = 4

NC = 2
NS = 16
CH = 128
NCH = 160
NPH = NCH // 2
E_PAD = NS * NCH * CH
RPT = 640
A_ROWS = NS * RPT

BN = 2000

NBUF = 4


def _trans_body(h_ref, wt_ref, b_ref, out_ref):
    res = (
        jnp.dot(h_ref[...], wt_ref[0], preferred_element_type=jnp.float32)
        + b_ref[0]
    )
    rb = res.astype(jnp.bfloat16)
    u = jax.lax.bitcast_convert_type(rb, jnp.uint16).astype(jnp.uint32)
    for half in range(NC):
        segs = []
        for g in range(2):
            base = HF * half + 32 * g
            lo = u[:, base:base + 16]
            hi = u[:, base + 16:base + 32]
            segs.append((hi << 16) | lo)
        packed = jnp.concatenate(segs, axis=1)
        out_ref[half, 0] = jax.lax.bitcast_convert_type(packed, jnp.int32)


def _trans_call(h, wt, b3):
    return pl.pallas_call(
        _trans_body,
        grid=(N_ETYPES, N_NODES // BN),
        in_specs=[
            pl.BlockSpec((BN, FEATS), lambda k, i: (i, 0)),
            pl.BlockSpec((1, FEATS, FEATS), lambda k, i: (k, 0, 0)),
            pl.BlockSpec((1, 1, FEATS), lambda k, i: (k, 0, 0)),
        ],
        out_specs=pl.BlockSpec((NC, 1, BN, PW), lambda k, i: (0, k, i, 0)),
        out_shape=jax.ShapeDtypeStruct(
            (NC, N_ETYPES, N_NODES, PW), jnp.int32),
    )(h, wt, b3)


def _sc_body(table_hbm, gidx_hbm, dst_hbm, zeros_hbm, out_hbm,
             gidx_v, dst_v, ibuf, fbuf, acc_sh, gsem, asem):
    c = lax.axis_index("c")
    s = lax.axis_index("s")
    my_table = table_hbm.at[c]

    pltpu.sync_copy(zeros_hbm.at[s], acc_sh.at[pl.ds(s * RPT, RPT)])
    plsc.subcore_barrier()

    def gather_start(q, t):
        pltpu.async_copy(my_table.at[gidx_v.at[q]], ibuf.at[t], gsem.at[t])

    def gather_wait(q, t):
        pltpu.make_async_copy(
            my_table.at[gidx_v.at[q]], ibuf.at[t], gsem.at[t]).wait()

    def add_start(q, t):
        pltpu.async_copy(fbuf.at[t & 1], acc_sh.at[dst_v.at[q]],
                         asem.at[t & 1], add=True)

    def add_wait(q, t):
        pltpu.make_async_copy(
            fbuf.at[t & 1], acc_sh.at[dst_v.at[q]], asem.at[t & 1]).wait()

    def unpack_chunk(t):
        p = t & 1

        def rows(r4, carry):
            for dr in range(4):
                r = r4 * 4 + dr
                for g in range(2):
                    x = ibuf[t, r, pl.ds(g * 16, 16)]
                    lo = jax.lax.bitcast_convert_type(x << 16, jnp.float32)
                    hi = jax.lax.bitcast_convert_type(
                        x & jnp.int32(-65536), jnp.float32)
                    fbuf[p, r, pl.ds(g * 32, 16)] = lo
                    fbuf[p, r, pl.ds(g * 32 + 16, 16)] = hi
            return carry

        lax.fori_loop(0, CH // 4, rows, 0)

    def process(q, t, guard_add):
        gather_wait(q, t)
        if guard_add:
            add_wait(q - 2, t)
        unpack_chunk(t)
        add_start(q, t)

    for phase in range(2):
        pltpu.sync_copy(gidx_hbm.at[phase, s], gidx_v)
        pltpu.sync_copy(dst_hbm.at[phase, s], dst_v)

        for t in range(NBUF):
            gather_start(t, t)
        for t in range(NBUF):
            process(t, t, guard_add=(t >= 2))
            gather_start(t + NBUF, t)

        def round_body(jj, carry):
            base = jj * NBUF
            for t in range(NBUF):
                process(base + t, t, guard_add=True)
                gather_start(base + NBUF + t, t)
            return carry

        lax.fori_loop(1, NPH // NBUF - 1, round_body, 0)

        base = NPH - NBUF
        for t in range(NBUF):
            process(base + t, t, guard_add=True)
        add_wait(NPH - 2, 0)
        add_wait(NPH - 1, 1)

    plsc.subcore_barrier()
    pltpu.sync_copy(acc_sh.at[pl.ds(s * RPT, RPT)], out_hbm.at[c, s])


@functools.cache
def _sc_aggregate_fn():
    mesh = plsc.VectorSubcoreMesh(core_axis_name="c", subcore_axis_name="s")
    return pl.kernel(
        _sc_body,
        mesh=mesh,
        compiler_params=pltpu.CompilerParams(use_tc_tiling_on_sc=False),
        out_type=jax.ShapeDtypeStruct((NC, NS, RPT, HF), jnp.float32),
        scratch_types=[
            pltpu.VMEM((NPH, CH), jnp.int32),
            pltpu.VMEM((NPH, CH), jnp.int32),
            pltpu.VMEM((NBUF, CH, PW), jnp.int32),
            pltpu.VMEM((2, CH, HF), jnp.float32),
            pltpu.VMEM_SHARED((A_ROWS, HF), jnp.float32),
            pltpu.SemaphoreType.DMA((NBUF,)),
            pltpu.SemaphoreType.DMA((2,)),
        ],
    )


def _gru_body(ap_ref, h_ref, wih_ref, whh_ref, bih_ref, bhh_ref, out_ref):
    a = jnp.concatenate([ap_ref[0], ap_ref[1]], axis=-1)
    h = h_ref[...]
    gi = jnp.dot(a, wih_ref[...], preferred_element_type=jnp.float32) + bih_ref[...]
    gh = jnp.dot(h, whh_ref[...], preferred_element_type=jnp.float32) + bhh_ref[...]
    r = jax.nn.sigmoid(gi[:, 0:FEATS] + gh[:, 0:FEATS])
    z = jax.nn.sigmoid(gi[:, FEATS:2 * FEATS] + gh[:, FEATS:2 * FEATS])
    n = jnp.tanh(gi[:, 2 * FEATS:] + r * gh[:, 2 * FEATS:])
    out_ref[...] = (1.0 - z) * n + z * h


def _gru_call(parts, h, wih_t, whh_t, bih2, bhh2):
    return pl.pallas_call(
        _gru_body,
        grid=(N_NODES // BN,),
        in_specs=[
            pl.BlockSpec((NC, BN, HF), lambda i: (0, i, 0)),
            pl.BlockSpec((BN, FEATS), lambda i: (i, 0)),
            pl.BlockSpec((FEATS, 3 * FEATS), lambda i: (0, 0)),
            pl.BlockSpec((FEATS, 3 * FEATS), lambda i: (0, 0)),
            pl.BlockSpec((1, 3 * FEATS), lambda i: (0, 0)),
            pl.BlockSpec((1, 3 * FEATS), lambda i: (0, 0)),
        ],
        out_specs=pl.BlockSpec((BN, FEATS), lambda i: (i, 0)),
        out_shape=jax.ShapeDtypeStruct((N_NODES, FEATS), jnp.float32),
    )(parts, h, wih_t, whh_t, bih2, bhh2)


def kernel(feat, edge_index, etypes, W_lin, b_lin, W_ih, W_hh, b_ih, b_hh):
    src = edge_index[0]
    dst = edge_index[1]

    pad = E_PAD - N_EDGES
    gidx = etypes * N_NODES + src
    gidx = jnp.concatenate([gidx, jnp.zeros((pad,), jnp.int32)])
    dstp = jnp.concatenate([dst, jnp.full((pad,), N_NODES, jnp.int32)])
    gidx4 = gidx.reshape(NS, 2, NPH, CH).transpose(1, 0, 2, 3)
    dst4 = dstp.reshape(NS, 2, NPH, CH).transpose(1, 0, 2, 3)
    zeros = jnp.zeros((NS, RPT, HF), jnp.float32)

    wt = W_lin.transpose(0, 2, 1)
    b3 = b_lin.reshape(N_ETYPES, 1, FEATS)
    wih_t = W_ih.T
    whh_t = W_hh.T
    bih2 = b_ih.reshape(1, 3 * FEATS)
    bhh2 = b_hh.reshape(1, 3 * FEATS)

    def step(h, _):
        trans = _trans_call(h, wt, b3)
        parts = _sc_aggregate_fn()(
            trans.reshape(NC, N_ETYPES * N_NODES, PW), gidx4, dst4, zeros)
        parts = parts.reshape(NC, A_ROWS, HF)
        h = _gru_call(parts, h, wih_t, whh_t, bih2, bhh2)
        return h, None

    h, _ = lax.scan(step, feat, None, length=N_STEPS)
    return h
